# baseline (speedup 1.0000x reference)
import functools
import numpy as np
import jax
import jax.numpy as jnp
from jax import lax
from jax.experimental import pallas as pl
from jax.experimental.pallas import tpu as pltpu
from jax.experimental.pallas import tpu_sc as plsc

N = 10000
E = 160000
D = 512
H = 8
C = 64

NC = 2
NS = 16
NW = NC * NS
EPW = 5008
EPAD = NW * EPW
SB = 16
NCHUNK = EPW // SB
NPAD = 10240
STRIPE = NPAD // NW
BINS = NW
CS = 64
REG = EPW + (BINS + 1) * CS
OFFW = 48

_MESH = dict(core_axis_name="c", subcore_axis_name="s", num_cores=NC,
             num_subcores=NS)
_SC_PARAMS = pltpu.CompilerParams(needs_layout_passes=False)


_BR = 400
_NBLK = N // _BR


def _qkvs_body(x_ref, w_ref, b_ref, sel_ref, y_ref, m_ref, mq_ref, mk_ref):
    i = pl.program_id(0)
    y = jnp.dot(x_ref[...], w_ref[...], preferred_element_type=jnp.float32)
    y = y + b_ref[...]
    y_ref[...] = y
    q = y[:, :D]
    k = y[:, D:2 * D]
    sel = sel_ref[...]
    hq = jnp.max(jnp.dot(q * q, sel, preferred_element_type=jnp.float32),
                 axis=0, keepdims=True)
    hk = jnp.max(jnp.dot(k * k, sel, preferred_element_type=jnp.float32),
                 axis=0, keepdims=True)

    @pl.when(i == 0)
    def _():
        mq_ref[...] = hq
        mk_ref[...] = hk

    @pl.when(i > 0)
    def _():
        mq_ref[...] = jnp.maximum(mq_ref[...], hq)
        mk_ref[...] = jnp.maximum(mk_ref[...], hk)

    @pl.when(i == _NBLK - 1)
    def _():
        m_ref[...] = jnp.sqrt(jnp.maximum(mq_ref[...] * mk_ref[...], 0.0)) / 8.0


def _qkvs(x, w_all, b_all, sel):
    y, m, _, _ = pl.pallas_call(
        _qkvs_body,
        grid=(_NBLK,),
        in_specs=[
            pl.BlockSpec((_BR, D), lambda i: (i, 0)),
            pl.BlockSpec((D, 4 * D), lambda i: (0, 0)),
            pl.BlockSpec((1, 4 * D), lambda i: (0, 0)),
            pl.BlockSpec((D, 128), lambda i: (0, 0)),
        ],
        out_specs=[
            pl.BlockSpec((_BR, 4 * D), lambda i: (i, 0)),
            pl.BlockSpec((1, 128), lambda i: (0, 0)),
            pl.BlockSpec((1, 128), lambda i: (0, 0)),
            pl.BlockSpec((1, 128), lambda i: (0, 0)),
        ],
        out_shape=[
            jax.ShapeDtypeStruct((N, 4 * D), jnp.float32),
            jax.ShapeDtypeStruct((1, 128), jnp.float32),
            jax.ShapeDtypeStruct((1, 128), jnp.float32),
            jax.ShapeDtypeStruct((1, 128), jnp.float32),
        ],
    )(x, w_all, b_all, sel)
    return y, m


def _sc_bin_body(src_hbm, dst_hbm,
                 bsrc_hbm, bdst_hbm, boffs_hbm, bcnt_hbm,
                 srcv, dstv, binv, bsv, bdv, offv, cntv):
    c = lax.axis_index("c")
    s = lax.axis_index("s")
    w = s * NC + c
    lanes = lax.iota(jnp.int32, 16)

    pltpu.sync_copy(src_hbm.at[pl.ds(w * EPW, EPW)], srcv)
    pltpu.sync_copy(dst_hbm.at[pl.ds(w * EPW, EPW)], dstv)

    def _zinit(i, _):
        zv = jnp.zeros((16,), jnp.int32)
        bsv[pl.ds(i * SB, SB)] = zv
        bdv[pl.ds(i * SB, SB)] = zv
        return 0

    lax.fori_loop(0, REG // SB, _zinit, 0)

    def _binc(i, _):
        d = dstv[pl.ds(i * SB, SB)]
        eid = w * EPW + i * SB + lanes
        binv[pl.ds(i * SB, SB)] = jnp.where(eid < E, d // STRIPE, BINS)
        return 0

    lax.fori_loop(0, NCHUNK, _binc, 0)

    offs = [jnp.zeros((16,), jnp.int32) for _ in range(3)]
    cnts = [jnp.zeros((16,), jnp.int32) for _ in range(3)]
    cur = jnp.int32(0)
    for b in range(BINS + 1):
        start = cur

        def _pass(i, cur):
            m = binv[pl.ds(i * SB, SB)] == b
            plsc.store_compressed(bsv.at[pl.ds(cur, SB)],
                                  srcv[pl.ds(i * SB, SB)], mask=m)
            plsc.store_compressed(bdv.at[pl.ds(cur, SB)],
                                  dstv[pl.ds(i * SB, SB)], mask=m)
            return cur + plsc.all_reduce_population_count(m)[0]

        cur = lax.fori_loop(0, NCHUNK, _pass, cur)
        kreg, lane = b // 16, b % 16
        offs[kreg] = jnp.where(lanes == lane, start, offs[kreg])
        cnts[kreg] = jnp.where(lanes == lane, cur - start, cnts[kreg])
        if b < BINS:
            npad = (-cur) % CS
            for ip in range(CS // SB):
                pm = lanes < npad - ip * SB
                plsc.store_compressed(bsv.at[pl.ds(cur + ip * SB, SB)],
                                      jnp.zeros((16,), jnp.int32), mask=pm)
                plsc.store_compressed(bdv.at[pl.ds(cur + ip * SB, SB)],
                                      jnp.full((16,), b * STRIPE, jnp.int32),
                                      mask=pm)
            cur = cur + npad

    for kreg in range(3):
        offv[pl.ds(kreg * 16, 16)] = offs[kreg]
        cntv[pl.ds(kreg * 16, 16)] = cnts[kreg]

    pltpu.sync_copy(bsv, bsrc_hbm.at[pl.ds(w * REG, REG)])
    pltpu.sync_copy(bdv, bdst_hbm.at[pl.ds(w * REG, REG)])
    pltpu.sync_copy(offv, boffs_hbm.at[pl.ds(w * OFFW, OFFW)])
    pltpu.sync_copy(cntv, bcnt_hbm.at[pl.ds(w * OFFW, OFFW)])


_sc_bin = functools.partial(
    pl.kernel,
    out_type=[
        jax.ShapeDtypeStruct((NW * REG,), jnp.int32),
        jax.ShapeDtypeStruct((NW * REG,), jnp.int32),
        jax.ShapeDtypeStruct((NW * OFFW,), jnp.int32),
        jax.ShapeDtypeStruct((NW * OFFW,), jnp.int32),
    ],
    mesh=plsc.VectorSubcoreMesh(**_MESH),
    compiler_params=_SC_PARAMS,
    scratch_types=[
        pltpu.VMEM((EPW,), jnp.int32),
        pltpu.VMEM((EPW,), jnp.int32),
        pltpu.VMEM((EPW,), jnp.int32),
        pltpu.VMEM((REG,), jnp.int32),
        pltpu.VMEM((REG,), jnp.int32),
        pltpu.VMEM((OFFW,), jnp.int32),
        pltpu.VMEM((OFFW,), jnp.int32),
    ],
)(_sc_bin_body)


def _sel48(tab_ref, row, col):
    win = (col // 16) * 16
    v = tab_ref[pl.ds(row * OFFW + win, 16)]
    return jnp.take(v, jnp.full((16,), col % 16, jnp.int32))[0]


def _sc_alpha_body(q_hbm, k_hbm, bsrc_hbm, bdst_hbm, boffs_hbm, bcnt_hbm,
                   m_hbm, ex_hbm, den_hbm,
                   qg, kg, ex_t, idx_s, idx_d, m_v, offt, cntt, den_l, sem):
    c = lax.axis_index("c")
    s = lax.axis_index("s")
    g = s * NC + c
    lanes = lax.iota(jnp.int32, 16)

    pltpu.sync_copy(m_hbm.at[pl.ds(0, 16)], m_v)
    mv = m_v[...]
    pltpu.sync_copy(boffs_hbm, offt)
    pltpu.sync_copy(bcnt_hbm, cntt)

    def _zrow(r, _):
        den_l[r, :] = jnp.zeros((16,), jnp.float32)
        return 0

    lax.fori_loop(0, STRIPE, _zrow, 0)

    def _wloop(wp, _):
        st = _sel48(offt, wp, g)
        cnt = _sel48(cntt, wp, g)

        def _chunk(t, _):
            off = pl.multiple_of(wp * REG + st + t * CS, SB)
            c1 = pltpu.async_copy(bsrc_hbm.at[pl.ds(off, CS)], idx_s, sem)
            c2 = pltpu.async_copy(bdst_hbm.at[pl.ds(off, CS)], idx_d, sem)
            c1.wait()
            c2.wait()
            c3 = pltpu.async_copy(q_hbm.at[idx_d], qg, sem)
            c4 = pltpu.async_copy(k_hbm.at[idx_s], kg, sem)
            c3.wait()
            c4.wait()
            vlim = cnt - t * CS

            def _sub(i4, _):
                dvec = idx_d[pl.ds(i4 * SB, SB)]
                vl = vlim - i4 * SB
                for e in range(SB):
                    av = jnp.zeros((16,), jnp.float32)
                    for h in range(H):
                        sh = jnp.zeros((16,), jnp.float32)
                        for j in range(4):
                            o = h * C + j * 16
                            sh = (sh + qg[i4 * SB + e, pl.ds(o, 16)]
                                  * kg[i4 * SB + e, pl.ds(o, 16)])
                        av = jnp.where(lanes == h, jnp.sum(sh), av)
                    lim = jnp.where(e < vl, 8, 0)
                    ex = jnp.where(lanes < lim, jnp.exp(av * 0.125 - mv), 0.0)
                    ex_t[e, :] = ex
                    dl = jnp.where(e < vl, dvec[e] - g * STRIPE, 0)
                    den_l[dl, :] = den_l[dl, :] + ex

                pltpu.sync_copy(
                    ex_t, ex_hbm.at[pl.ds(
                        pl.multiple_of(off + i4 * SB, SB), SB)])
                return 0

            lax.fori_loop(0, CS // SB, _sub, 0)
            return 0

        lax.fori_loop(0, (cnt + CS - 1) // CS, _chunk, 0)
        return 0

    lax.fori_loop(0, NW, _wloop, 0)
    pltpu.sync_copy(den_l, den_hbm.at[pl.ds(g * STRIPE, STRIPE)])


_sc_alpha = functools.partial(
    pl.kernel,
    out_type=[
        jax.ShapeDtypeStruct((NW * REG, 16), jnp.float32),
        jax.ShapeDtypeStruct((NPAD, 16), jnp.float32),
    ],
    mesh=plsc.VectorSubcoreMesh(**_MESH),
    compiler_params=_SC_PARAMS,
    scratch_types=[
        pltpu.VMEM((CS, D), jnp.float32),
        pltpu.VMEM((CS, D), jnp.float32),
        pltpu.VMEM((SB, 16), jnp.float32),
        pltpu.VMEM((CS,), jnp.int32),
        pltpu.VMEM((CS,), jnp.int32),
        pltpu.VMEM((16,), jnp.float32),
        pltpu.VMEM((NW * OFFW,), jnp.int32),
        pltpu.VMEM((NW * OFFW,), jnp.int32),
        pltpu.VMEM((STRIPE, 16), jnp.float32),
        pltpu.SemaphoreType.DMA,
    ],
)(_sc_alpha_body)


def _sc_agg_body(vt0, vt1, vt2, vt3, bsrc_hbm, bdst_hbm, boffs_hbm, bcnt_hbm,
                 ex_hbm, agg_hbm,
                 vg, exb, idx_s, idx_d, offt, cntt, agg_l, sem):
    c = lax.axis_index("c")
    s = lax.axis_index("s")
    g = s * NC + c

    pltpu.sync_copy(boffs_hbm, offt)
    pltpu.sync_copy(bcnt_hbm, cntt)

    for p in range(4):
        vt = (vt0, vt1, vt2, vt3)[p]

        def _zrow(r, _):
            for j in range(8):
                agg_l[r, pl.ds(j * 16, 16)] = jnp.zeros((16,), jnp.float32)
            return 0

        lax.fori_loop(0, STRIPE, _zrow, 0)

        def _wloop(wp, _):
            st = _sel48(offt, wp, g)
            cnt = _sel48(cntt, wp, g)

            def _chunk(t, _):
                off = pl.multiple_of(wp * REG + st + t * CS, SB)
                c1 = pltpu.async_copy(bsrc_hbm.at[pl.ds(off, CS)], idx_s, sem)
                c2 = pltpu.async_copy(bdst_hbm.at[pl.ds(off, CS)], idx_d, sem)
                c3 = pltpu.async_copy(ex_hbm.at[pl.ds(off, CS)], exb, sem)
                c1.wait()
                c2.wait()
                c3.wait()
                pltpu.async_copy(vt.at[idx_s], vg, sem).wait()
                vlim = cnt - t * CS

                def _sub(i4, _):
                    dvec = idx_d[pl.ds(i4 * SB, SB)]
                    vl = vlim - i4 * SB
                    for e in range(SB):
                        exv = exb[i4 * SB + e, :]
                        valid = e < vl
                        e0 = jnp.where(valid, exv[2 * p], 0.0)
                        e1 = jnp.where(valid, exv[2 * p + 1], 0.0)
                        dl = jnp.where(valid, dvec[e] - g * STRIPE, 0)
                        for j in range(4):
                            sl = pl.ds(j * 16, 16)
                            agg_l[dl, sl] = (agg_l[dl, sl]
                                             + vg[i4 * SB + e, sl] * e0)
                        for j in range(4, 8):
                            sl = pl.ds(j * 16, 16)
                            agg_l[dl, sl] = (agg_l[dl, sl]
                                             + vg[i4 * SB + e, sl] * e1)
                    return 0

                lax.fori_loop(0, CS // SB, _sub, 0)
                return 0

            lax.fori_loop(0, (cnt + CS - 1) // CS, _chunk, 0)
            return 0

        lax.fori_loop(0, NW, _wloop, 0)
        pltpu.sync_copy(agg_l, agg_hbm.at[p].at[pl.ds(g * STRIPE, STRIPE)])


_sc_agg = functools.partial(
    pl.kernel,
    out_type=jax.ShapeDtypeStruct((4, NPAD, 128), jnp.float32),
    mesh=plsc.VectorSubcoreMesh(**_MESH),
    compiler_params=_SC_PARAMS,
    scratch_types=[
        pltpu.VMEM((CS, 128), jnp.float32),
        pltpu.VMEM((CS, 16), jnp.float32),
        pltpu.VMEM((CS,), jnp.int32),
        pltpu.VMEM((CS,), jnp.int32),
        pltpu.VMEM((NW * OFFW,), jnp.int32),
        pltpu.VMEM((NW * OFFW,), jnp.int32),
        pltpu.VMEM((STRIPE, 128), jnp.float32),
        pltpu.SemaphoreType.DMA,
    ],
)(_sc_agg_body)


def _merge_body(a_ref, d_ref, skip_ref, res_ref, o_ref):
    den = d_ref[0]
    den_b = jnp.concatenate(
        [jnp.broadcast_to(den[:, h:h + 1], (_BR, C)) for h in range(H)],
        axis=1)
    agg = jnp.concatenate([a_ref[0, j] for j in range(4)], axis=1)
    o_ref[...] = agg / (den_b + 1e-16) + skip_ref[...] + res_ref[...]


def _merge(agg, den, skip, res):
    return pl.pallas_call(
        _merge_body,
        grid=(_NBLK,),
        in_specs=[
            pl.BlockSpec((1, 4, _BR, 128), lambda i: (0, 0, i, 0)),
            pl.BlockSpec((1, _BR, 16), lambda i: (0, i, 0)),
            pl.BlockSpec((_BR, D), lambda i: (i, 0)),
            pl.BlockSpec((_BR, D), lambda i: (i, 0)),
        ],
        out_specs=pl.BlockSpec((_BR, D), lambda i: (i, 0)),
        out_shape=jax.ShapeDtypeStruct((N, D), jnp.float32),
    )(agg[None], den[None], skip, res)


def _ln(x, g, b):
    m = jnp.mean(x, axis=-1, keepdims=True)
    v = jnp.mean((x - m) ** 2, axis=-1, keepdims=True)
    return (x - m) / jnp.sqrt(v + 1e-5) * g + b


def _ffn_body(x_ref, enc_ref, w1_ref, b1_ref, w2_ref, b2_ref, g_ref, bb_ref,
              o_ref):
    g = g_ref[...]
    bb = bb_ref[...]
    t = _ln(x_ref[...], g, bb)
    h1 = jnp.maximum(
        jnp.dot(t, w1_ref[...], preferred_element_type=jnp.float32)
        + b1_ref[...], 0.0)
    y = (jnp.dot(h1, w2_ref[...], preferred_element_type=jnp.float32)
         + b2_ref[...] + t)
    o_ref[...] = _ln(y, g, bb) + enc_ref[...]


def _ffn(x, enc, w1t, b1, w2t, b2, g, bb):
    return pl.pallas_call(
        _ffn_body,
        grid=(_NBLK,),
        in_specs=[
            pl.BlockSpec((_BR, D), lambda i: (i, 0)),
            pl.BlockSpec((_BR, D), lambda i: (i, 0)),
            pl.BlockSpec((D, 2 * D), lambda i: (0, 0)),
            pl.BlockSpec((1, 2 * D), lambda i: (0, 0)),
            pl.BlockSpec((2 * D, D), lambda i: (0, 0)),
            pl.BlockSpec((1, D), lambda i: (0, 0)),
            pl.BlockSpec((1, D), lambda i: (0, 0)),
            pl.BlockSpec((1, D), lambda i: (0, 0)),
        ],
        out_specs=pl.BlockSpec((_BR, D), lambda i: (i, 0)),
        out_shape=jax.ShapeDtypeStruct((N, D), jnp.float32),
    )(x, enc, w1t, b1, w2t, b2, g, bb)


_SEL = np.zeros((D, 128), np.float32)
for _h in range(H):
    _SEL[_h * C:(_h + 1) * C, _h] = 1.0


def _conv_layer(x, binned, w_all, b_all, res):
    bsrc, bdst, boffs, bcnt = binned
    y, m = _qkvs(x, w_all, b_all, _SEL)
    v = y[:, 2 * D:3 * D]
    skip = y[:, 3 * D:]
    ex, den = _sc_alpha(y[:, :D], y[:, D:2 * D], bsrc, bdst, boffs, bcnt,
                        m.reshape(-1))
    vts = [v[:, p * 128:(p + 1) * 128] for p in range(4)]
    agg = _sc_agg(vts[0], vts[1], vts[2], vts[3], bsrc, bdst, boffs, bcnt, ex)
    return _merge(agg, den, skip, res)


def kernel(x, edge_index, edge_type, edge_repre, Wq0, bq0, Wk0, bk0, Wv0, bv0,
           Ws0, bs0, Wq1, bq1, Wk1, bk1, Wv1, bv1, Ws1, bs1, W1, b1, W2, b2,
           ln_g, ln_b):
    del edge_type, edge_repre
    pad = EPAD - E
    srcp = jnp.pad(edge_index[0], (0, pad))
    dstp = jnp.pad(edge_index[1], (0, pad))
    binned = _sc_bin(srcp, dstp)

    w_all0 = jnp.concatenate([Wq0.T, Wk0.T, Wv0.T, Ws0.T], axis=1)
    b_all0 = jnp.concatenate([bq0, bk0, bv0, bs0]).reshape(1, -1)
    w_all1 = jnp.concatenate([Wq1.T, Wk1.T, Wv1.T, Ws1.T], axis=1)
    b_all1 = jnp.concatenate([bq1, bk1, bv1, bs1]).reshape(1, -1)

    enc = x
    x1 = _conv_layer(enc, binned, w_all0, b_all0, enc)
    x2 = _conv_layer(x1, binned, w_all1, b_all1, enc)
    return _ffn(x2, enc, W1.T, b1.reshape(1, -1), W2.T, b2.reshape(1, -1),
                ln_g.reshape(1, -1), ln_b.reshape(1, -1))


# speedup vs baseline: 1.3997x; 1.3997x over previous
import functools
import numpy as np
import jax
import jax.numpy as jnp
from jax import lax
from jax.experimental import pallas as pl
from jax.experimental.pallas import tpu as pltpu
from jax.experimental.pallas import tpu_sc as plsc

N = 10000
E = 160000
D = 512
H = 8
C = 64

NC = 2
NS = 16
NW = NC * NS
EPW = 5008
EPAD = NW * EPW
SB = 16
NCHUNK = EPW // SB
NPAD = 10240
STRIPE = NPAD // NW
BINS = NW
CS = 64
REG = EPW + (BINS + 1) * CS
OFFW = 48

_MESH = dict(core_axis_name="c", subcore_axis_name="s", num_cores=NC,
             num_subcores=NS)
_SC_PARAMS = pltpu.CompilerParams(needs_layout_passes=False)


_BR = 400
_NBLK = N // _BR


def _qkvs_body(x_ref, w_ref, b_ref, sel_ref, y_ref, m_ref, mq_ref, mk_ref):
    i = pl.program_id(0)
    y = jnp.dot(x_ref[...], w_ref[...], preferred_element_type=jnp.float32)
    y = y + b_ref[...]
    y_ref[...] = y
    q = y[:, :D]
    k = y[:, D:2 * D]
    sel = sel_ref[...]
    hq = jnp.max(jnp.dot(q * q, sel, preferred_element_type=jnp.float32),
                 axis=0, keepdims=True)
    hk = jnp.max(jnp.dot(k * k, sel, preferred_element_type=jnp.float32),
                 axis=0, keepdims=True)

    @pl.when(i == 0)
    def _():
        mq_ref[...] = hq
        mk_ref[...] = hk

    @pl.when(i > 0)
    def _():
        mq_ref[...] = jnp.maximum(mq_ref[...], hq)
        mk_ref[...] = jnp.maximum(mk_ref[...], hk)

    @pl.when(i == _NBLK - 1)
    def _():
        m_ref[...] = jnp.sqrt(jnp.maximum(mq_ref[...] * mk_ref[...], 0.0)) / 8.0


def _qkvs(x, w_all, b_all, sel):
    y, m, _, _ = pl.pallas_call(
        _qkvs_body,
        grid=(_NBLK,),
        in_specs=[
            pl.BlockSpec((_BR, D), lambda i: (i, 0)),
            pl.BlockSpec((D, 4 * D), lambda i: (0, 0)),
            pl.BlockSpec((1, 4 * D), lambda i: (0, 0)),
            pl.BlockSpec((D, 128), lambda i: (0, 0)),
        ],
        out_specs=[
            pl.BlockSpec((_BR, 4 * D), lambda i: (i, 0)),
            pl.BlockSpec((1, 128), lambda i: (0, 0)),
            pl.BlockSpec((1, 128), lambda i: (0, 0)),
            pl.BlockSpec((1, 128), lambda i: (0, 0)),
        ],
        out_shape=[
            jax.ShapeDtypeStruct((N, 4 * D), jnp.float32),
            jax.ShapeDtypeStruct((1, 128), jnp.float32),
            jax.ShapeDtypeStruct((1, 128), jnp.float32),
            jax.ShapeDtypeStruct((1, 128), jnp.float32),
        ],
    )(x, w_all, b_all, sel)
    return y, m


def _sc_bin_body(src_hbm, dst_hbm,
                 bsrc_hbm, bdst_hbm, boffs_hbm, bcnt_hbm,
                 srcv, dstv, binv, bsv, bdv, offv, cntv):
    c = lax.axis_index("c")
    s = lax.axis_index("s")
    w = s * NC + c
    lanes = lax.iota(jnp.int32, 16)

    pltpu.sync_copy(src_hbm.at[pl.ds(w * EPW, EPW)], srcv)
    pltpu.sync_copy(dst_hbm.at[pl.ds(w * EPW, EPW)], dstv)

    def _zinit(i, _):
        zv = jnp.zeros((16,), jnp.int32)
        bsv[pl.ds(i * SB, SB)] = zv
        bdv[pl.ds(i * SB, SB)] = zv
        return 0

    lax.fori_loop(0, REG // SB, _zinit, 0)

    def _binc(i, _):
        d = dstv[pl.ds(i * SB, SB)]
        eid = w * EPW + i * SB + lanes
        binv[pl.ds(i * SB, SB)] = jnp.where(eid < E, d // STRIPE, BINS)
        return 0

    lax.fori_loop(0, NCHUNK, _binc, 0)

    offs = [jnp.zeros((16,), jnp.int32) for _ in range(3)]
    cnts = [jnp.zeros((16,), jnp.int32) for _ in range(3)]
    cur = jnp.int32(0)
    for b in range(BINS + 1):
        start = cur

        def _pass(i, cur):
            m = binv[pl.ds(i * SB, SB)] == b
            plsc.store_compressed(bsv.at[pl.ds(cur, SB)],
                                  srcv[pl.ds(i * SB, SB)], mask=m)
            plsc.store_compressed(bdv.at[pl.ds(cur, SB)],
                                  dstv[pl.ds(i * SB, SB)], mask=m)
            return cur + plsc.all_reduce_population_count(m)[0]

        cur = lax.fori_loop(0, NCHUNK, _pass, cur)
        kreg, lane = b // 16, b % 16
        offs[kreg] = jnp.where(lanes == lane, start, offs[kreg])
        cnts[kreg] = jnp.where(lanes == lane, cur - start, cnts[kreg])
        if b < BINS:
            npad = (-cur) % CS
            for ip in range(CS // SB):
                pm = lanes < npad - ip * SB
                plsc.store_compressed(bsv.at[pl.ds(cur + ip * SB, SB)],
                                      jnp.zeros((16,), jnp.int32), mask=pm)
                plsc.store_compressed(bdv.at[pl.ds(cur + ip * SB, SB)],
                                      jnp.full((16,), b * STRIPE, jnp.int32),
                                      mask=pm)
            cur = cur + npad

    for kreg in range(3):
        offv[pl.ds(kreg * 16, 16)] = offs[kreg]
        cntv[pl.ds(kreg * 16, 16)] = cnts[kreg]

    pltpu.sync_copy(bsv, bsrc_hbm.at[pl.ds(w * REG, REG)])
    pltpu.sync_copy(bdv, bdst_hbm.at[pl.ds(w * REG, REG)])
    pltpu.sync_copy(offv, boffs_hbm.at[pl.ds(w * OFFW, OFFW)])
    pltpu.sync_copy(cntv, bcnt_hbm.at[pl.ds(w * OFFW, OFFW)])


_sc_bin = functools.partial(
    pl.kernel,
    out_type=[
        jax.ShapeDtypeStruct((NW * REG,), jnp.int32),
        jax.ShapeDtypeStruct((NW * REG,), jnp.int32),
        jax.ShapeDtypeStruct((NW * OFFW,), jnp.int32),
        jax.ShapeDtypeStruct((NW * OFFW,), jnp.int32),
    ],
    mesh=plsc.VectorSubcoreMesh(**_MESH),
    compiler_params=_SC_PARAMS,
    scratch_types=[
        pltpu.VMEM((EPW,), jnp.int32),
        pltpu.VMEM((EPW,), jnp.int32),
        pltpu.VMEM((EPW,), jnp.int32),
        pltpu.VMEM((REG,), jnp.int32),
        pltpu.VMEM((REG,), jnp.int32),
        pltpu.VMEM((OFFW,), jnp.int32),
        pltpu.VMEM((OFFW,), jnp.int32),
    ],
)(_sc_bin_body)


def _sel48(tab_ref, row, col):
    win = (col // 16) * 16
    v = tab_ref[pl.ds(row * OFFW + win, 16)]
    return jnp.take(v, jnp.full((16,), col % 16, jnp.int32))[0]


def _sc_alpha_body(q_hbm, k_hbm, bsrc_hbm, bdst_hbm, boffs_hbm, bcnt_hbm,
                   m_hbm, ex_hbm, den_hbm,
                   qg, kg, ex_t, idx_s, idx_d, m_v, offt, cntt, den_l, sem):
    c = lax.axis_index("c")
    s = lax.axis_index("s")
    g = s * NC + c
    lanes = lax.iota(jnp.int32, 16)

    pltpu.sync_copy(m_hbm.at[pl.ds(0, 16)], m_v)
    mv = m_v[...]
    pltpu.sync_copy(boffs_hbm, offt)
    pltpu.sync_copy(bcnt_hbm, cntt)

    def _zrow(r, _):
        den_l[r, :] = jnp.zeros((16,), jnp.float32)
        return 0

    lax.fori_loop(0, STRIPE, _zrow, 0)

    def _wloop(wp, _):
        st = _sel48(offt, wp, g)
        cnt = _sel48(cntt, wp, g)

        def _chunk(t, _):
            off = pl.multiple_of(wp * REG + st + t * CS, SB)
            c1 = pltpu.async_copy(bsrc_hbm.at[pl.ds(off, CS)], idx_s, sem)
            c2 = pltpu.async_copy(bdst_hbm.at[pl.ds(off, CS)], idx_d, sem)
            c1.wait()
            c2.wait()
            c3 = pltpu.async_copy(q_hbm.at[idx_d], qg, sem)
            c4 = pltpu.async_copy(k_hbm.at[idx_s], kg, sem)
            c3.wait()
            c4.wait()
            vlim = cnt - t * CS

            def _sub(i4, _):
                dvec = idx_d[pl.ds(i4 * SB, SB)]
                vl = vlim - i4 * SB
                for e in range(SB):
                    av = jnp.zeros((16,), jnp.float32)
                    for h in range(H):
                        sh = jnp.zeros((16,), jnp.float32)
                        for j in range(4):
                            o = h * C + j * 16
                            sh = (sh + qg[i4 * SB + e, pl.ds(o, 16)]
                                  * kg[i4 * SB + e, pl.ds(o, 16)])
                        av = jnp.where(lanes == h, jnp.sum(sh), av)
                    lim = jnp.where(e < vl, 8, 0)
                    ex = jnp.where(lanes < lim, jnp.exp(av * 0.125 - mv), 0.0)
                    ex_t[e, :] = ex
                    dl = jnp.where(e < vl, dvec[e] - g * STRIPE, 0)
                    den_l[dl, :] = den_l[dl, :] + ex

                pltpu.sync_copy(
                    ex_t, ex_hbm.at[pl.ds(
                        pl.multiple_of(off + i4 * SB, SB), SB)])
                return 0

            lax.fori_loop(0, CS // SB, _sub, 0)
            return 0

        lax.fori_loop(0, (cnt + CS - 1) // CS, _chunk, 0)
        return 0

    lax.fori_loop(0, NW, _wloop, 0)
    pltpu.sync_copy(den_l, den_hbm.at[pl.ds(g * STRIPE, STRIPE)])


_sc_alpha = functools.partial(
    pl.kernel,
    out_type=[
        jax.ShapeDtypeStruct((NW * REG, 16), jnp.float32),
        jax.ShapeDtypeStruct((NPAD, 16), jnp.float32),
    ],
    mesh=plsc.VectorSubcoreMesh(**_MESH),
    compiler_params=_SC_PARAMS,
    scratch_types=[
        pltpu.VMEM((CS, D), jnp.float32),
        pltpu.VMEM((CS, D), jnp.float32),
        pltpu.VMEM((SB, 16), jnp.float32),
        pltpu.VMEM((CS,), jnp.int32),
        pltpu.VMEM((CS,), jnp.int32),
        pltpu.VMEM((16,), jnp.float32),
        pltpu.VMEM((NW * OFFW,), jnp.int32),
        pltpu.VMEM((NW * OFFW,), jnp.int32),
        pltpu.VMEM((STRIPE, 16), jnp.float32),
        pltpu.SemaphoreType.DMA,
    ],
)(_sc_alpha_body)


def _sc_agg_body(vt0, vt1, bsrc_hbm, bdst_hbm, boffs_hbm, bcnt_hbm,
                 ex_hbm, agg_hbm,
                 vg, exb, idx_s, idx_d, offt, cntt, agg_l, sem):
    c = lax.axis_index("c")
    s = lax.axis_index("s")
    g = s * NC + c

    pltpu.sync_copy(boffs_hbm, offt)
    pltpu.sync_copy(bcnt_hbm, cntt)

    for p in range(2):
        vt = (vt0, vt1)[p]

        def _zrow(r, _):
            for j in range(16):
                agg_l[r, pl.ds(j * 16, 16)] = jnp.zeros((16,), jnp.float32)
            return 0

        lax.fori_loop(0, STRIPE, _zrow, 0)

        def _wloop(wp, _):
            st = _sel48(offt, wp, g)
            cnt = _sel48(cntt, wp, g)

            def _chunk(t, _):
                off = pl.multiple_of(wp * REG + st + t * CS, SB)
                c1 = pltpu.async_copy(bsrc_hbm.at[pl.ds(off, CS)], idx_s, sem)
                c2 = pltpu.async_copy(bdst_hbm.at[pl.ds(off, CS)], idx_d, sem)
                c3 = pltpu.async_copy(ex_hbm.at[pl.ds(off, CS)], exb, sem)
                c1.wait()
                c2.wait()
                c3.wait()
                pltpu.async_copy(vt.at[idx_s], vg, sem).wait()
                vlim = cnt - t * CS

                def _sub(i4, _):
                    dvec = idx_d[pl.ds(i4 * SB, SB)]
                    vl = vlim - i4 * SB
                    for e in range(SB):
                        exv = exb[i4 * SB + e, :]
                        valid = e < vl
                        es = [jnp.where(valid, exv[4 * p + jh], 0.0)
                              for jh in range(4)]
                        dl = jnp.where(valid, dvec[e] - g * STRIPE, 0)
                        for j in range(16):
                            sl = pl.ds(j * 16, 16)
                            agg_l[dl, sl] = (agg_l[dl, sl]
                                             + vg[i4 * SB + e, sl] * es[j // 4])
                    return 0

                lax.fori_loop(0, CS // SB, _sub, 0)
                return 0

            lax.fori_loop(0, (cnt + CS - 1) // CS, _chunk, 0)
            return 0

        lax.fori_loop(0, NW, _wloop, 0)
        pltpu.sync_copy(agg_l, agg_hbm.at[p].at[pl.ds(g * STRIPE, STRIPE)])


_sc_agg = functools.partial(
    pl.kernel,
    out_type=jax.ShapeDtypeStruct((2, NPAD, 256), jnp.float32),
    mesh=plsc.VectorSubcoreMesh(**_MESH),
    compiler_params=_SC_PARAMS,
    scratch_types=[
        pltpu.VMEM((CS, 256), jnp.float32),
        pltpu.VMEM((CS, 16), jnp.float32),
        pltpu.VMEM((CS,), jnp.int32),
        pltpu.VMEM((CS,), jnp.int32),
        pltpu.VMEM((NW * OFFW,), jnp.int32),
        pltpu.VMEM((NW * OFFW,), jnp.int32),
        pltpu.VMEM((STRIPE, 256), jnp.float32),
        pltpu.SemaphoreType.DMA,
    ],
)(_sc_agg_body)


def _merge_body(a_ref, d_ref, skip_ref, res_ref, o_ref):
    den = d_ref[0]
    den_b = jnp.concatenate(
        [jnp.broadcast_to(den[:, h:h + 1], (_BR, C)) for h in range(H)],
        axis=1)
    agg = jnp.concatenate([a_ref[0, j] for j in range(2)], axis=1)
    o_ref[...] = agg / (den_b + 1e-16) + skip_ref[...] + res_ref[...]


def _merge(agg, den, skip, res):
    return pl.pallas_call(
        _merge_body,
        grid=(_NBLK,),
        in_specs=[
            pl.BlockSpec((1, 2, _BR, 256), lambda i: (0, 0, i, 0)),
            pl.BlockSpec((1, _BR, 16), lambda i: (0, i, 0)),
            pl.BlockSpec((_BR, D), lambda i: (i, 0)),
            pl.BlockSpec((_BR, D), lambda i: (i, 0)),
        ],
        out_specs=pl.BlockSpec((_BR, D), lambda i: (i, 0)),
        out_shape=jax.ShapeDtypeStruct((N, D), jnp.float32),
    )(agg[None], den[None], skip, res)


def _ln(x, g, b):
    m = jnp.mean(x, axis=-1, keepdims=True)
    v = jnp.mean((x - m) ** 2, axis=-1, keepdims=True)
    return (x - m) / jnp.sqrt(v + 1e-5) * g + b


def _ffn_body(x_ref, enc_ref, w1_ref, b1_ref, w2_ref, b2_ref, g_ref, bb_ref,
              o_ref):
    g = g_ref[...]
    bb = bb_ref[...]
    t = _ln(x_ref[...], g, bb)
    h1 = jnp.maximum(
        jnp.dot(t, w1_ref[...], preferred_element_type=jnp.float32)
        + b1_ref[...], 0.0)
    y = (jnp.dot(h1, w2_ref[...], preferred_element_type=jnp.float32)
         + b2_ref[...] + t)
    o_ref[...] = _ln(y, g, bb) + enc_ref[...]


def _ffn(x, enc, w1t, b1, w2t, b2, g, bb):
    return pl.pallas_call(
        _ffn_body,
        grid=(_NBLK,),
        in_specs=[
            pl.BlockSpec((_BR, D), lambda i: (i, 0)),
            pl.BlockSpec((_BR, D), lambda i: (i, 0)),
            pl.BlockSpec((D, 2 * D), lambda i: (0, 0)),
            pl.BlockSpec((1, 2 * D), lambda i: (0, 0)),
            pl.BlockSpec((2 * D, D), lambda i: (0, 0)),
            pl.BlockSpec((1, D), lambda i: (0, 0)),
            pl.BlockSpec((1, D), lambda i: (0, 0)),
            pl.BlockSpec((1, D), lambda i: (0, 0)),
        ],
        out_specs=pl.BlockSpec((_BR, D), lambda i: (i, 0)),
        out_shape=jax.ShapeDtypeStruct((N, D), jnp.float32),
    )(x, enc, w1t, b1, w2t, b2, g, bb)


_SEL = np.zeros((D, 128), np.float32)
for _h in range(H):
    _SEL[_h * C:(_h + 1) * C, _h] = 1.0


def _conv_layer(x, binned, w_all, b_all, res):
    bsrc, bdst, boffs, bcnt = binned
    y, m = _qkvs(x, w_all, b_all, _SEL)
    v = y[:, 2 * D:3 * D]
    skip = y[:, 3 * D:]
    ex, den = _sc_alpha(y[:, :D], y[:, D:2 * D], bsrc, bdst, boffs, bcnt,
                        m.reshape(-1))
    agg = _sc_agg(v[:, :2 * C * 2], v[:, 2 * C * 2:], bsrc, bdst, boffs,
                  bcnt, ex)
    return _merge(agg, den, skip, res)


def kernel(x, edge_index, edge_type, edge_repre, Wq0, bq0, Wk0, bk0, Wv0, bv0,
           Ws0, bs0, Wq1, bq1, Wk1, bk1, Wv1, bv1, Ws1, bs1, W1, b1, W2, b2,
           ln_g, ln_b):
    del edge_type, edge_repre
    pad = EPAD - E
    srcp = jnp.pad(edge_index[0], (0, pad))
    dstp = jnp.pad(edge_index[1], (0, pad))
    binned = _sc_bin(srcp, dstp)

    w_all0 = jnp.concatenate([Wq0.T, Wk0.T, Wv0.T, Ws0.T], axis=1)
    b_all0 = jnp.concatenate([bq0, bk0, bv0, bs0]).reshape(1, -1)
    w_all1 = jnp.concatenate([Wq1.T, Wk1.T, Wv1.T, Ws1.T], axis=1)
    b_all1 = jnp.concatenate([bq1, bk1, bv1, bs1]).reshape(1, -1)

    enc = x
    x1 = _conv_layer(enc, binned, w_all0, b_all0, enc)
    x2 = _conv_layer(x1, binned, w_all1, b_all1, enc)
    return _ffn(x2, enc, W1.T, b1.reshape(1, -1), W2.T, b2.reshape(1, -1),
                ln_g.reshape(1, -1), ln_b.reshape(1, -1))
